# baseline (speedup 1.0000x reference)
import functools

import jax
import jax.numpy as jnp
from jax import lax
from jax.experimental import pallas as pl
from jax.experimental.pallas import tpu as pltpu
from jax.experimental.pallas import tpu_sc as plsc

N = 10000
E = 320000
H = 128
NC = 2
NS = 16
NW = NC * NS
PER_W = E // NW
CH = 80
N_CH = PER_W // CH
ST = 624
REM = N - NS * ST

_sc_mesh = plsc.VectorSubcoreMesh(core_axis_name="c", subcore_axis_name="s",
                                  num_cores=NC, num_subcores=NS)


PCH = 128
PN_CH = 80
PAD_W = PCH * PN_CH
ACC_R = N + NS


def _pad_indices(src, dst):
    s2 = src.reshape(NW, PER_W)
    d2 = dst.reshape(NW, PER_W)
    padn = PAD_W - PER_W
    spad = jnp.zeros((NW, padn), jnp.int32)
    dpadv = (N + (jnp.arange(NW, dtype=jnp.int32) % NS))[:, None]
    srcp = jnp.concatenate([s2, spad], axis=1).reshape(NW * PN_CH, PCH)
    dstp = jnp.concatenate([d2, jnp.broadcast_to(dpadv, (NW, padn))],
                           axis=1).reshape(NW * PN_CH, PCH)
    return srcp, dstp


def _acc_init(z_hbm, acc, sid):
    pltpu.sync_copy(z_hbm.at[pl.ds(sid * ST, ST)], acc.at[pl.ds(sid * ST, ST)])

    @pl.when(sid == NS - 1)
    def _():
        pltpu.sync_copy(z_hbm.at[pl.ds(NS * ST, REM)],
                        acc.at[pl.ds(NS * ST, REM)])


def _acc_dump(acc, out_hbm, cid, sid):
    pltpu.sync_copy(acc.at[pl.ds(sid * ST, ST)],
                    out_hbm.at[pl.ds(cid * N + sid * ST, ST)])

    @pl.when(sid == NS - 1)
    def _():
        pltpu.sync_copy(acc.at[pl.ds(NS * ST, REM)],
                        out_hbm.at[pl.ds(cid * N + NS * ST, REM)])


GRP = 8
N_GRP = PN_CH // GRP


def _sc_segsum(t, srcp, dstp, zeros):
    @functools.partial(
        pl.kernel,
        out_type=jax.ShapeDtypeStruct((NC * N, H), jnp.float32),
        mesh=_sc_mesh,
        scratch_types=(
            [pltpu.VMEM((PN_CH, PCH), jnp.int32)]
            + [pltpu.VMEM((GRP, PCH), jnp.int32)] * 2
            + [pltpu.VMEM((PCH, H), jnp.float32)] * 2
            + [pltpu.VMEM_SHARED((ACC_R, H), jnp.float32)]
            + [pltpu.SemaphoreType.DMA] * 6
        ),
    )
    def k(t_hbm, srcp_hbm, dstp_hbm, z_hbm, out_hbm, didx2, ib0, ib1,
          r0, r1, acc, sI0, sI1, sg0, sg1, ss0, ss1):
        idxb = (ib0, ib1)
        rows = (r0, r1)
        sI = (sI0, sI1)
        sg = (sg0, sg1)
        ss = (ss0, ss1)
        cid = lax.axis_index("c")
        sid = lax.axis_index("s")
        wid = cid * NS + sid
        ibase = wid * PN_CH

        pltpu.sync_copy(dstp_hbm.at[pl.ds(ibase, PN_CH)], didx2)
        _acc_init(z_hbm, acc, sid)

        def fire_idx(m, mb):
            pltpu.async_copy(srcp_hbm.at[pl.ds(ibase + m * GRP, GRP)],
                             idxb[mb], sI[mb])

        def wait_idx(mb):
            pltpu.make_async_copy(srcp_hbm.at[pl.ds(ibase, GRP)], idxb[mb],
                                  sI[mb]).wait()

        def fire_gather(r, mb, b, j):
            del j
            pltpu.async_copy(t_hbm.at[idxb[mb].at[r]], rows[b], sg[b])

        def fire_scatter(j, b):
            pltpu.async_copy(rows[b], acc.at[didx2.at[j]], ss[b], add=True)

        def wait_gather(b):
            pltpu.make_async_copy(t_hbm.at[idxb[0].at[0]], rows[b],
                                  sg[b]).wait()

        def wait_scatter(b):
            pltpu.make_async_copy(rows[b], acc.at[didx2.at[0]], ss[b]).wait()

        fire_idx(0, 0)
        fire_idx(1, 1)
        plsc.subcore_barrier()

        wait_idx(0)
        fire_gather(0, 0, 0, 0)
        wait_gather(0)
        fire_scatter(0, 0)
        fire_gather(1, 0, 1, 1)
        for r in range(2, GRP):
            b = r % 2
            wait_gather(1 - b)
            fire_scatter(r - 1, 1 - b)
            wait_scatter(b)
            fire_gather(r, 0, b, r)

        def group_body(m, mb, fire_next):
            wait_gather(1)
            fire_scatter(8 * m - 1, 1)
            if fire_next:
                fire_idx(m + 1, 1 - mb)
            wait_idx(mb)
            wait_scatter(0)
            fire_gather(0, mb, 0, None)
            for r in range(1, GRP):
                b = r % 2
                wait_gather(1 - b)
                fire_scatter(8 * m + r - 1, 1 - b)
                wait_scatter(b)
                fire_gather(r, mb, b, None)

        group_body(1, 1, fire_next=True)

        @pl.loop(1, (N_GRP - 2) // 2)
        def _(G):
            group_body(2 * G, 0, fire_next=True)
            group_body(2 * G + 1, 1, fire_next=True)

        group_body(N_GRP - 2, 0, fire_next=True)
        group_body(N_GRP - 1, 1, fire_next=False)

        wait_gather(1)
        fire_scatter(PN_CH - 1, 1)
        wait_scatter(0)
        wait_scatter(1)

        plsc.subcore_barrier()
        _acc_dump(acc, out_hbm, cid, sid)

    return k(t, srcp, dstp, zeros)


NB2 = 4


def _sc_gather2(a, b, src, dst):
    @functools.partial(
        pl.kernel,
        out_type=[jax.ShapeDtypeStruct((E, H), jnp.float32),
                  jax.ShapeDtypeStruct((E, H), jnp.float32)],
        mesh=_sc_mesh,
        scratch_types=(
            [pltpu.VMEM((PER_W,), jnp.int32)] * 2
            + [pltpu.VMEM((CH, H), jnp.float32)] * (2 * NB2)
            + [pltpu.SemaphoreType.DMA] * (2 * NB2)
        ),
    )
    def k(a_hbm, b_hbm, src_hbm, dst_hbm, r1_hbm, r2_hbm, *scr):
        sidx, didx = scr[0], scr[1]
        r1b = scr[2:2 + NB2]
        r2b = scr[2 + NB2:2 + 2 * NB2]
        sg = scr[2 + 2 * NB2:2 + 3 * NB2]
        sw = scr[2 + 3 * NB2:]
        cid = lax.axis_index("c")
        sid = lax.axis_index("s")
        base = (cid * NS + sid) * PER_W
        pltpu.sync_copy(src_hbm.at[pl.ds(base, PER_W)], sidx)
        pltpu.sync_copy(dst_hbm.at[pl.ds(base, PER_W)], didx)

        def fire_gathers(j, bb):
            pltpu.async_copy(a_hbm.at[sidx.at[pl.ds(j * CH, CH)]], r1b[bb],
                             sg[bb])
            pltpu.async_copy(b_hbm.at[didx.at[pl.ds(j * CH, CH)]], r2b[bb],
                             sg[bb])

        def wait_gathers(bb):
            pltpu.make_async_copy(a_hbm.at[sidx.at[pl.ds(0, CH)]], r1b[bb],
                                  sg[bb]).wait()
            pltpu.make_async_copy(b_hbm.at[didx.at[pl.ds(0, CH)]], r2b[bb],
                                  sg[bb]).wait()

        def fire_writes(j, bb):
            pltpu.async_copy(r1b[bb], r1_hbm.at[pl.ds(base + j * CH, CH)],
                             sw[bb])
            pltpu.async_copy(r2b[bb], r2_hbm.at[pl.ds(base + j * CH, CH)],
                             sw[bb])

        def wait_writes(bb):
            pltpu.make_async_copy(r1b[bb], r1_hbm.at[pl.ds(base, CH)],
                                  sw[bb]).wait()
            pltpu.make_async_copy(r2b[bb], r2_hbm.at[pl.ds(base, CH)],
                                  sw[bb]).wait()

        for j in range(NB2):
            if j >= 2:
                wait_gathers(j - 2)
                fire_writes(j - 2, j - 2)
            fire_gathers(j, j)

        @pl.loop(1, N_CH // NB2)
        def _(g):
            c0 = g * NB2
            for bb in range(NB2):
                j = c0 + bb
                bp = (bb - 2) % NB2
                wait_gathers(bp)
                fire_writes(j - 2, bp)
                wait_writes(bb)
                fire_gathers(j, bb)

        j = N_CH - 1
        wait_gathers((j - 2) % NB2)
        fire_writes(j - 2, (j - 2) % NB2)
        wait_writes(j % NB2)
        fire_gathers(j, j % NB2)
        for q in (N_CH - 2, N_CH - 1):
            wait_gathers(q % NB2)
            fire_writes(q, q % NB2)
        for q in range(N_CH - NB2, N_CH):
            wait_writes(q % NB2)

    return k(a, b, src, dst)


_RB = 2000
_EB = 2000


def _ln_relu(h, s, b):
    mu = jnp.mean(h, axis=-1, keepdims=True)
    d = h - mu
    var = jnp.mean(d * d, axis=-1, keepdims=True)
    return jnp.maximum(d * lax.rsqrt(var + 1e-5) * s + b, 0.0)


def _tc_encode(x, W_enc, b_enc, s0, b0):
    def body(x_ref, w_ref, be_ref, s_ref, b_ref, h_ref, t_ref):
        h = jnp.dot(x_ref[...], w_ref[...],
                    preferred_element_type=jnp.float32) + be_ref[...]
        h_ref[...] = h
        t_ref[...] = _ln_relu(h, s_ref[...], b_ref[...])

    full = pl.BlockSpec((H, H), lambda i: (0, 0))
    vec = pl.BlockSpec((1, H), lambda i: (0, 0))
    rows = pl.BlockSpec((_RB, H), lambda i: (i, 0))
    return pl.pallas_call(
        body,
        grid=(N // _RB,),
        in_specs=[rows, full, vec, vec, vec],
        out_specs=[rows, rows],
        out_shape=[jax.ShapeDtypeStruct((N, H), jnp.float32),
                   jax.ShapeDtypeStruct((N, H), jnp.float32)],
    )(x, W_enc, b_enc, s0, b0)


def _agg_from_parts(acc_ref, cnt_ref):
    a = acc_ref[0] + acc_ref[1]
    cnt = cnt_ref[0, :, :1] + cnt_ref[1, :, :1]
    inv = 1.0 / jnp.maximum(cnt, 1.0)
    eps = 1e-7 * (cnt > 0.0).astype(jnp.float32)
    return a * inv + eps


def _tc_layer(acc, cnt, h, Wl_i, bl_i, s_next, b_next):
    def body(acc_ref, cnt_ref, h_ref, w_ref, bv_ref, s_ref, b_ref,
             h_out, t_out):
        agg = _agg_from_parts(acc_ref, cnt_ref)
        hn = h_ref[...] + jnp.dot(agg, w_ref[...],
                                  preferred_element_type=jnp.float32) + bv_ref[...]
        h_out[...] = hn
        t_out[...] = _ln_relu(hn, s_ref[...], b_ref[...])

    rows = pl.BlockSpec((_RB, H), lambda i: (i, 0))
    acc_spec = pl.BlockSpec((NC, _RB, H), lambda i: (0, i, 0))
    cnt_spec = pl.BlockSpec((NC, _RB, H), lambda i: (0, i, 0))
    full = pl.BlockSpec((H, H), lambda i: (0, 0))
    vec = pl.BlockSpec((1, H), lambda i: (0, 0))
    return pl.pallas_call(
        body,
        grid=(N // _RB,),
        in_specs=[acc_spec, cnt_spec, rows, full, vec, vec, vec],
        out_specs=[rows, rows],
        out_shape=[jax.ShapeDtypeStruct((N, H), jnp.float32),
                   jax.ShapeDtypeStruct((N, H), jnp.float32)],
    )(acc, cnt, h, Wl_i, bl_i, s_next, b_next)


def _tc_final_proj(acc, cnt, h, Wl_i, bl_i, sf, bf, W1a, W1b, b1):
    def body(acc_ref, cnt_ref, h_ref, w_ref, bv_ref, s_ref, b_ref,
             w1a_ref, w1b_ref, b1_ref, a_out, b_out):
        agg = _agg_from_parts(acc_ref, cnt_ref)
        hn = h_ref[...] + jnp.dot(agg, w_ref[...],
                                  preferred_element_type=jnp.float32) + bv_ref[...]
        hf = _ln_relu(hn, s_ref[...], b_ref[...])
        a_out[...] = jnp.dot(hf, w1a_ref[...],
                             preferred_element_type=jnp.float32) + b1_ref[...]
        b_out[...] = jnp.dot(hf, w1b_ref[...],
                             preferred_element_type=jnp.float32)

    rows = pl.BlockSpec((_RB, H), lambda i: (i, 0))
    acc_spec = pl.BlockSpec((NC, _RB, H), lambda i: (0, i, 0))
    cnt_spec = pl.BlockSpec((NC, _RB, H), lambda i: (0, i, 0))
    full = pl.BlockSpec((H, H), lambda i: (0, 0))
    vec = pl.BlockSpec((1, H), lambda i: (0, 0))
    return pl.pallas_call(
        body,
        grid=(N // _RB,),
        in_specs=[acc_spec, cnt_spec, rows, full, vec, vec, vec,
                  full, full, vec],
        out_specs=[rows, rows],
        out_shape=[jax.ShapeDtypeStruct((N, H), jnp.float32),
                   jax.ShapeDtypeStruct((N, H), jnp.float32)],
    )(acc, cnt, h, Wl_i, bl_i, sf, bf, W1a, W1b, b1)


def _tc_mlp(r1, r2, W2, b2):
    def body(r1_ref, r2_ref, w_ref, bv_ref, o_ref):
        r = jnp.maximum(r1_ref[...] + r2_ref[...], 0.0)
        o_ref[...] = jnp.dot(r, w_ref[...],
                             preferred_element_type=jnp.float32) + bv_ref[...]

    rows = pl.BlockSpec((_EB, H), lambda i: (i, 0))
    full = pl.BlockSpec((H, H), lambda i: (0, 0))
    vec = pl.BlockSpec((1, H), lambda i: (0, 0))
    return pl.pallas_call(
        body,
        grid=(E // _EB,),
        in_specs=[rows, rows, full, vec],
        out_specs=rows,
        out_shape=jax.ShapeDtypeStruct((E, H), jnp.float32),
    )(r1, r2, W2, b2)


def kernel(x, edge_index, W_enc, b_enc, Wl, bl, ln_s, ln_b, lnf_s, lnf_b,
           W1, b1, W2, b2):
    L = Wl.shape[0]
    src = edge_index[0]
    dst = edge_index[1]
    zeros = jnp.zeros((N, H), jnp.float32)
    ones_tab = jnp.ones((N, H), jnp.float32)
    srcp, dstp = _pad_indices(src, dst)
    row = lambda v: v.reshape(1, -1)

    cnt = _sc_segsum(ones_tab, jnp.zeros_like(srcp), dstp, zeros).reshape(NC, N, H)
    h, t = _tc_encode(x, W_enc, row(b_enc), row(ln_s[0]), row(ln_b[0]))
    for i in range(L):
        acc = _sc_segsum(t, srcp, dstp, zeros).reshape(NC, N, H)
        if i + 1 < L:
            h, t = _tc_layer(acc, cnt, h, Wl[i], row(bl[i]),
                             row(ln_s[i + 1]), row(ln_b[i + 1]))
        else:
            a_tab, b_tab = _tc_final_proj(acc, cnt, h, Wl[i], row(bl[i]),
                                          row(lnf_s), row(lnf_b),
                                          W1[:H], W1[H:], row(b1))
    r1, r2 = _sc_gather2(a_tab, b_tab, src, dst)
    return _tc_mlp(r1, r2, W2, row(b2))


# speedup vs baseline: 5.8527x; 5.8527x over previous
import functools

import jax
import jax.numpy as jnp
from jax import lax
from jax.experimental import pallas as pl
from jax.experimental.pallas import tpu as pltpu
from jax.experimental.pallas import tpu_sc as plsc

N = 10000
E = 320000
H = 128
NC = 2
NS = 16
NW = NC * NS
PER_W = E // NW
CH = 80
N_CH = PER_W // CH
ST = 624
REM = N - NS * ST

_sc_mesh = plsc.VectorSubcoreMesh(core_axis_name="c", subcore_axis_name="s",
                                  num_cores=NC, num_subcores=NS)


PCH = 128
PN_CH = 80
PAD_W = PCH * PN_CH
ACC_R = N + NS


def _pad_indices(src, dst):
    s2 = src.reshape(NW, PER_W)
    d2 = dst.reshape(NW, PER_W)
    padn = PAD_W - PER_W
    spad = jnp.zeros((NW, padn), jnp.int32)
    dpadv = (N + (jnp.arange(NW, dtype=jnp.int32) % NS))[:, None]
    srcp = jnp.concatenate([s2, spad], axis=1).reshape(NW * PAD_W)
    dstp = jnp.concatenate([d2, jnp.broadcast_to(dpadv, (NW, padn))],
                           axis=1).reshape(NW * PAD_W)
    return srcp, dstp


def _acc_init(z_hbm, acc, sid):
    pltpu.sync_copy(z_hbm.at[pl.ds(sid * ST, ST)], acc.at[pl.ds(sid * ST, ST)])

    @pl.when(sid == NS - 1)
    def _():
        pltpu.sync_copy(z_hbm.at[pl.ds(NS * ST, REM)],
                        acc.at[pl.ds(NS * ST, REM)])


def _acc_dump(acc, out_hbm, cid, sid):
    pltpu.sync_copy(acc.at[pl.ds(sid * ST, ST)],
                    out_hbm.at[pl.ds(cid * N + sid * ST, ST)])

    @pl.when(sid == NS - 1)
    def _():
        pltpu.sync_copy(acc.at[pl.ds(NS * ST, REM)],
                        out_hbm.at[pl.ds(cid * N + NS * ST, REM)])


def _sc_segsum(t, srcp, dstp, zeros):
    @functools.partial(
        pl.kernel,
        out_type=jax.ShapeDtypeStruct((NC * N, H), jnp.float32),
        mesh=_sc_mesh,
        scratch_types=(
            [pltpu.VMEM((PCH,), jnp.int32)] * 8
            + [pltpu.VMEM((PCH, H), jnp.float32)] * 2
            + [pltpu.VMEM_SHARED((ACC_R, H), jnp.float32)]
            + [pltpu.SemaphoreType.DMA] * 10
        ),
    )
    def k(t_hbm, srcp_hbm, dstp_hbm, z_hbm, out_hbm, *scr):
        sidx = scr[0:4]
        didx = scr[4:8]
        rows = scr[8:10]
        acc = scr[10]
        sI = scr[11:15]
        sg = scr[15:17]
        ss = scr[17:21]
        cid = lax.axis_index("c")
        sid = lax.axis_index("s")
        wid = cid * NS + sid
        ibase = wid * PAD_W

        _acc_init(z_hbm, acc, sid)

        def fire_idx(q, s4):
            pltpu.async_copy(srcp_hbm.at[pl.ds(ibase + q * PCH, PCH)],
                             sidx[s4], sI[s4])
            pltpu.async_copy(dstp_hbm.at[pl.ds(ibase + q * PCH, PCH)],
                             didx[s4], sI[s4])

        def wait_idx(s4):
            pltpu.make_async_copy(srcp_hbm.at[pl.ds(0, PCH)], sidx[s4],
                                  sI[s4]).wait()
            pltpu.make_async_copy(dstp_hbm.at[pl.ds(0, PCH)], didx[s4],
                                  sI[s4]).wait()

        def fire_gather(s4, s2):
            pltpu.async_copy(t_hbm.at[sidx[s4]], rows[s2], sg[s2])

        def wait_gather(s2):
            pltpu.make_async_copy(t_hbm.at[sidx[0]], rows[s2], sg[s2]).wait()

        def fire_scatter(s4, s2):
            pltpu.async_copy(rows[s2], acc.at[didx[s4]], ss[s4], add=True)

        def wait_scatter(s4, s2):
            pltpu.make_async_copy(rows[s2], acc.at[didx[0]], ss[s4]).wait()

        fire_idx(0, 0)
        fire_idx(1, 1)
        plsc.subcore_barrier()

        def slot(c, q_next, steady):
            s4, s2 = c % 4, c % 2
            wait_idx(s4)
            if steady:
                wait_scatter((s4 + 2) % 4, s2)
            fire_gather(s4, s2)
            if c >= 1:
                wait_gather(1 - s2)
                fire_scatter((s4 - 1) % 4, 1 - s2)
            if q_next is not None:
                fire_idx(q_next, (s4 + 2) % 4)

        slot(0, 2, steady=False)
        slot(1, 3, steady=False)
        slot(2, 4, steady=True)
        slot(3, 5, steady=True)

        @pl.loop(0, 18)
        def _(g):
            c0 = 4 * g + 4
            for b in range(4):
                s2 = b % 2
                wait_idx(b)
                wait_scatter((b + 2) % 4, s2)
                fire_gather(b, s2)
                wait_gather(1 - s2)
                fire_scatter((b - 1) % 4, 1 - s2)
                pltpu.async_copy(
                    srcp_hbm.at[pl.ds(ibase + (c0 + b + 2) * PCH, PCH)],
                    sidx[(b + 2) % 4], sI[(b + 2) % 4])
                pltpu.async_copy(
                    dstp_hbm.at[pl.ds(ibase + (c0 + b + 2) * PCH, PCH)],
                    didx[(b + 2) % 4], sI[(b + 2) % 4])

        slot(76, 78, steady=True)
        slot(77, 79, steady=True)
        slot(78, None, steady=True)
        slot(79, None, steady=True)
        wait_gather(1)
        fire_scatter(3, 1)
        wait_scatter(2, 0)
        wait_scatter(3, 1)

        plsc.subcore_barrier()
        _acc_dump(acc, out_hbm, cid, sid)

    return k(t, srcp, dstp, zeros)


NB2 = 4


def _sc_gather2(a, b, src, dst):
    @functools.partial(
        pl.kernel,
        out_type=[jax.ShapeDtypeStruct((E, H), jnp.float32),
                  jax.ShapeDtypeStruct((E, H), jnp.float32)],
        mesh=_sc_mesh,
        scratch_types=(
            [pltpu.VMEM((PER_W,), jnp.int32)] * 2
            + [pltpu.VMEM((CH, H), jnp.float32)] * (2 * NB2)
            + [pltpu.SemaphoreType.DMA] * (2 * NB2)
        ),
    )
    def k(a_hbm, b_hbm, src_hbm, dst_hbm, r1_hbm, r2_hbm, *scr):
        sidx, didx = scr[0], scr[1]
        r1b = scr[2:2 + NB2]
        r2b = scr[2 + NB2:2 + 2 * NB2]
        sg = scr[2 + 2 * NB2:2 + 3 * NB2]
        sw = scr[2 + 3 * NB2:]
        cid = lax.axis_index("c")
        sid = lax.axis_index("s")
        base = (cid * NS + sid) * PER_W
        pltpu.sync_copy(src_hbm.at[pl.ds(base, PER_W)], sidx)
        pltpu.sync_copy(dst_hbm.at[pl.ds(base, PER_W)], didx)

        def fire_gathers(j, bb):
            pltpu.async_copy(a_hbm.at[sidx.at[pl.ds(j * CH, CH)]], r1b[bb],
                             sg[bb])
            pltpu.async_copy(b_hbm.at[didx.at[pl.ds(j * CH, CH)]], r2b[bb],
                             sg[bb])

        def wait_gathers(bb):
            pltpu.make_async_copy(a_hbm.at[sidx.at[pl.ds(0, CH)]], r1b[bb],
                                  sg[bb]).wait()
            pltpu.make_async_copy(b_hbm.at[didx.at[pl.ds(0, CH)]], r2b[bb],
                                  sg[bb]).wait()

        def fire_writes(j, bb):
            pltpu.async_copy(r1b[bb], r1_hbm.at[pl.ds(base + j * CH, CH)],
                             sw[bb])
            pltpu.async_copy(r2b[bb], r2_hbm.at[pl.ds(base + j * CH, CH)],
                             sw[bb])

        def wait_writes(bb):
            pltpu.make_async_copy(r1b[bb], r1_hbm.at[pl.ds(base, CH)],
                                  sw[bb]).wait()
            pltpu.make_async_copy(r2b[bb], r2_hbm.at[pl.ds(base, CH)],
                                  sw[bb]).wait()

        for j in range(NB2):
            if j >= 2:
                wait_gathers(j - 2)
                fire_writes(j - 2, j - 2)
            fire_gathers(j, j)

        @pl.loop(1, N_CH // NB2)
        def _(g):
            c0 = g * NB2
            for bb in range(NB2):
                j = c0 + bb
                bp = (bb - 2) % NB2
                wait_gathers(bp)
                fire_writes(j - 2, bp)
                wait_writes(bb)
                fire_gathers(j, bb)

        j = N_CH - 1
        wait_gathers((j - 2) % NB2)
        fire_writes(j - 2, (j - 2) % NB2)
        wait_writes(j % NB2)
        fire_gathers(j, j % NB2)
        for q in (N_CH - 2, N_CH - 1):
            wait_gathers(q % NB2)
            fire_writes(q, q % NB2)
        for q in range(N_CH - NB2, N_CH):
            wait_writes(q % NB2)

    return k(a, b, src, dst)


_RB = 2000
_EB = 2000


def _ln_relu(h, s, b):
    mu = jnp.mean(h, axis=-1, keepdims=True)
    d = h - mu
    var = jnp.mean(d * d, axis=-1, keepdims=True)
    return jnp.maximum(d * lax.rsqrt(var + 1e-5) * s + b, 0.0)


def _tc_encode(x, W_enc, b_enc, s0, b0):
    def body(x_ref, w_ref, be_ref, s_ref, b_ref, h_ref, t_ref):
        h = jnp.dot(x_ref[...], w_ref[...],
                    preferred_element_type=jnp.float32) + be_ref[...]
        h_ref[...] = h
        t_ref[...] = _ln_relu(h, s_ref[...], b_ref[...])

    full = pl.BlockSpec((H, H), lambda i: (0, 0))
    vec = pl.BlockSpec((1, H), lambda i: (0, 0))
    rows = pl.BlockSpec((_RB, H), lambda i: (i, 0))
    return pl.pallas_call(
        body,
        grid=(N // _RB,),
        in_specs=[rows, full, vec, vec, vec],
        out_specs=[rows, rows],
        out_shape=[jax.ShapeDtypeStruct((N, H), jnp.float32),
                   jax.ShapeDtypeStruct((N, H), jnp.float32)],
    )(x, W_enc, b_enc, s0, b0)


def _agg_from_parts(acc_ref, cnt_ref):
    a = acc_ref[0] + acc_ref[1]
    cnt = cnt_ref[0, :, :1] + cnt_ref[1, :, :1]
    inv = 1.0 / jnp.maximum(cnt, 1.0)
    eps = 1e-7 * (cnt > 0.0).astype(jnp.float32)
    return a * inv + eps


def _tc_layer(acc, cnt, h, Wl_i, bl_i, s_next, b_next):
    def body(acc_ref, cnt_ref, h_ref, w_ref, bv_ref, s_ref, b_ref,
             h_out, t_out):
        agg = _agg_from_parts(acc_ref, cnt_ref)
        hn = h_ref[...] + jnp.dot(agg, w_ref[...],
                                  preferred_element_type=jnp.float32) + bv_ref[...]
        h_out[...] = hn
        t_out[...] = _ln_relu(hn, s_ref[...], b_ref[...])

    rows = pl.BlockSpec((_RB, H), lambda i: (i, 0))
    acc_spec = pl.BlockSpec((NC, _RB, H), lambda i: (0, i, 0))
    cnt_spec = pl.BlockSpec((NC, _RB, H), lambda i: (0, i, 0))
    full = pl.BlockSpec((H, H), lambda i: (0, 0))
    vec = pl.BlockSpec((1, H), lambda i: (0, 0))
    return pl.pallas_call(
        body,
        grid=(N // _RB,),
        in_specs=[acc_spec, cnt_spec, rows, full, vec, vec, vec],
        out_specs=[rows, rows],
        out_shape=[jax.ShapeDtypeStruct((N, H), jnp.float32),
                   jax.ShapeDtypeStruct((N, H), jnp.float32)],
    )(acc, cnt, h, Wl_i, bl_i, s_next, b_next)


def _tc_final_proj(acc, cnt, h, Wl_i, bl_i, sf, bf, W1a, W1b, b1):
    def body(acc_ref, cnt_ref, h_ref, w_ref, bv_ref, s_ref, b_ref,
             w1a_ref, w1b_ref, b1_ref, a_out, b_out):
        agg = _agg_from_parts(acc_ref, cnt_ref)
        hn = h_ref[...] + jnp.dot(agg, w_ref[...],
                                  preferred_element_type=jnp.float32) + bv_ref[...]
        hf = _ln_relu(hn, s_ref[...], b_ref[...])
        a_out[...] = jnp.dot(hf, w1a_ref[...],
                             preferred_element_type=jnp.float32) + b1_ref[...]
        b_out[...] = jnp.dot(hf, w1b_ref[...],
                             preferred_element_type=jnp.float32)

    rows = pl.BlockSpec((_RB, H), lambda i: (i, 0))
    acc_spec = pl.BlockSpec((NC, _RB, H), lambda i: (0, i, 0))
    cnt_spec = pl.BlockSpec((NC, _RB, H), lambda i: (0, i, 0))
    full = pl.BlockSpec((H, H), lambda i: (0, 0))
    vec = pl.BlockSpec((1, H), lambda i: (0, 0))
    return pl.pallas_call(
        body,
        grid=(N // _RB,),
        in_specs=[acc_spec, cnt_spec, rows, full, vec, vec, vec,
                  full, full, vec],
        out_specs=[rows, rows],
        out_shape=[jax.ShapeDtypeStruct((N, H), jnp.float32),
                   jax.ShapeDtypeStruct((N, H), jnp.float32)],
    )(acc, cnt, h, Wl_i, bl_i, sf, bf, W1a, W1b, b1)


def _tc_mlp(r1, r2, W2, b2):
    def body(r1_ref, r2_ref, w_ref, bv_ref, o_ref):
        r = jnp.maximum(r1_ref[...] + r2_ref[...], 0.0)
        o_ref[...] = jnp.dot(r, w_ref[...],
                             preferred_element_type=jnp.float32) + bv_ref[...]

    rows = pl.BlockSpec((_EB, H), lambda i: (i, 0))
    full = pl.BlockSpec((H, H), lambda i: (0, 0))
    vec = pl.BlockSpec((1, H), lambda i: (0, 0))
    return pl.pallas_call(
        body,
        grid=(E // _EB,),
        in_specs=[rows, rows, full, vec],
        out_specs=rows,
        out_shape=jax.ShapeDtypeStruct((E, H), jnp.float32),
    )(r1, r2, W2, b2)


def kernel(x, edge_index, W_enc, b_enc, Wl, bl, ln_s, ln_b, lnf_s, lnf_b,
           W1, b1, W2, b2):
    L = Wl.shape[0]
    src = edge_index[0]
    dst = edge_index[1]
    zeros = jnp.zeros((N, H), jnp.float32)
    ones_tab = jnp.ones((N, H), jnp.float32)
    srcp, dstp = _pad_indices(src, dst)
    row = lambda v: v.reshape(1, -1)

    cnt = _sc_segsum(ones_tab, srcp, dstp, zeros).reshape(NC, N, H)
    h, t = _tc_encode(x, W_enc, row(b_enc), row(ln_s[0]), row(ln_b[0]))
    for i in range(L):
        acc = _sc_segsum(t, srcp, dstp, zeros).reshape(NC, N, H)
        if i + 1 < L:
            h, t = _tc_layer(acc, cnt, h, Wl[i], row(bl[i]),
                             row(ln_s[i + 1]), row(ln_b[i + 1]))
        else:
            a_tab, b_tab = _tc_final_proj(acc, cnt, h, Wl[i], row(bl[i]),
                                          row(lnf_s), row(lnf_b),
                                          W1[:H], W1[H:], row(b1))
    r1, r2 = _sc_gather2(a_tab, b_tab, src, dst)
    return _tc_mlp(r1, r2, W2, row(b2))


# speedup vs baseline: 10.0718x; 1.7209x over previous
import functools

import jax
import jax.numpy as jnp
from jax import lax
from jax.experimental import pallas as pl
from jax.experimental.pallas import tpu as pltpu
from jax.experimental.pallas import tpu_sc as plsc

N = 10000
E = 320000
H = 128
NC = 2
NS = 16
NW = NC * NS
PER_W = E // NW
CH = 80
N_CH = PER_W // CH
ST = 624
REM = N - NS * ST

_sc_mesh = plsc.VectorSubcoreMesh(core_axis_name="c", subcore_axis_name="s",
                                  num_cores=NC, num_subcores=NS)


ACC_R = N + NS


def _acc_init(z_hbm, acc, sid):
    pltpu.sync_copy(z_hbm.at[pl.ds(sid * ST, ST)], acc.at[pl.ds(sid * ST, ST)])

    @pl.when(sid == NS - 1)
    def _():
        pltpu.sync_copy(z_hbm.at[pl.ds(NS * ST, REM)],
                        acc.at[pl.ds(NS * ST, REM)])


def _acc_dump(acc, out_hbm, cid, sid):
    pltpu.sync_copy(acc.at[pl.ds(sid * ST, ST)],
                    out_hbm.at[pl.ds(cid * N + sid * ST, ST)])

    @pl.when(sid == NS - 1)
    def _():
        pltpu.sync_copy(acc.at[pl.ds(NS * ST, REM)],
                        out_hbm.at[pl.ds(cid * N + NS * ST, REM)])


def _sc_segsum(t, src, dst, zeros):
    @functools.partial(
        pl.kernel,
        out_type=jax.ShapeDtypeStruct((NC * N, H), jnp.float32),
        mesh=_sc_mesh,
        scratch_types=(
            [pltpu.VMEM((CH,), jnp.int32)] * 4
            + [pltpu.VMEM((CH, H), jnp.float32)] * 2
            + [pltpu.VMEM_SHARED((ACC_R, H), jnp.float32)]
            + [pltpu.SemaphoreType.DMA] * 2
        ),
    )
    def k(t_hbm, src_hbm, dst_hbm, z_hbm, out_hbm,
          si0, si1, di0, di1, r0, r1, acc, sg0, sg1):
        sidx = (si0, si1)
        didx = (di0, di1)
        rows = (r0, r1)
        sg = (sg0, sg1)
        cid = lax.axis_index("c")
        sid = lax.axis_index("s")
        _acc_init(z_hbm, acc, sid)
        plsc.subcore_barrier()
        base = (cid * NS + sid) * PER_W

        def loadidx(c, b):
            pltpu.sync_copy(src_hbm.at[pl.ds(base + c * CH, CH)], sidx[b])
            pltpu.sync_copy(dst_hbm.at[pl.ds(base + c * CH, CH)], didx[b])

        def wait_g(b):
            pltpu.make_async_copy(t_hbm.at[sidx[0]], rows[b], sg[b]).wait()

        loadidx(0, 0)
        pltpu.async_copy(t_hbm.at[sidx[0]], rows[0], sg[0])

        @pl.loop(0, (N_CH - 1) // 2)
        def _(g):
            for b in range(2):
                c = 2 * g + b
                loadidx(c + 1, 1 - b)
                pltpu.async_copy(t_hbm.at[sidx[1 - b]], rows[1 - b],
                                 sg[1 - b])
                wait_g(b)
                pltpu.sync_copy(rows[b], acc.at[didx[b]], add=True)

        wait_g(0)
        pltpu.sync_copy(rows[0], acc.at[didx[0]], add=True)
        plsc.subcore_barrier()
        _acc_dump(acc, out_hbm, cid, sid)

    return k(t, src, dst, zeros)


def _sc_count(dst, ones, zeros):
    @functools.partial(
        pl.kernel,
        out_type=jax.ShapeDtypeStruct((NC * N, H), jnp.float32),
        mesh=_sc_mesh,
        scratch_types=[
            pltpu.VMEM((CH,), jnp.int32),
            pltpu.VMEM((CH, H), jnp.float32),
            pltpu.VMEM_SHARED((ACC_R, H), jnp.float32),
        ],
    )
    def k(dst_hbm, ones_hbm, z_hbm, out_hbm, didx, ones_v, acc):
        cid = lax.axis_index("c")
        sid = lax.axis_index("s")
        pltpu.sync_copy(ones_hbm, ones_v)
        _acc_init(z_hbm, acc, sid)
        plsc.subcore_barrier()
        base = (cid * NS + sid) * PER_W

        @pl.loop(0, N_CH)
        def _(c):
            pltpu.sync_copy(dst_hbm.at[pl.ds(base + c * CH, CH)], didx)
            pltpu.sync_copy(ones_v, acc.at[didx], add=True)

        plsc.subcore_barrier()
        _acc_dump(acc, out_hbm, cid, sid)

    return k(dst, ones, zeros)


def _sc_gather2(a, b, src, dst):
    @functools.partial(
        pl.kernel,
        out_type=[jax.ShapeDtypeStruct((E, H), jnp.float32),
                  jax.ShapeDtypeStruct((E, H), jnp.float32)],
        mesh=_sc_mesh,
        scratch_types=(
            [pltpu.VMEM((CH,), jnp.int32)] * 4
            + [pltpu.VMEM((CH, H), jnp.float32)] * 4
            + [pltpu.SemaphoreType.DMA] * 2
        ),
    )
    def k(a_hbm, b_hbm, src_hbm, dst_hbm, r1_hbm, r2_hbm,
          si0, si1, di0, di1, p0, p1, q0, q1, sg0, sg1):
        sidx = (si0, si1)
        didx = (di0, di1)
        r1b = (p0, p1)
        r2b = (q0, q1)
        sg = (sg0, sg1)
        cid = lax.axis_index("c")
        sid = lax.axis_index("s")
        base = (cid * NS + sid) * PER_W

        def loadidx(c, bb):
            pltpu.sync_copy(src_hbm.at[pl.ds(base + c * CH, CH)], sidx[bb])
            pltpu.sync_copy(dst_hbm.at[pl.ds(base + c * CH, CH)], didx[bb])

        def fire(bb):
            pltpu.async_copy(a_hbm.at[sidx[bb]], r1b[bb], sg[bb])
            pltpu.async_copy(b_hbm.at[didx[bb]], r2b[bb], sg[bb])

        def wait_g(bb):
            pltpu.make_async_copy(a_hbm.at[sidx[0]], r1b[bb], sg[bb]).wait()
            pltpu.make_async_copy(b_hbm.at[didx[0]], r2b[bb], sg[bb]).wait()

        loadidx(0, 0)
        fire(0)

        @pl.loop(0, (N_CH - 1) // 2)
        def _(g):
            for bb in range(2):
                c = 2 * g + bb
                loadidx(c + 1, 1 - bb)
                fire(1 - bb)
                wait_g(bb)
                pltpu.sync_copy(r1b[bb], r1_hbm.at[pl.ds(base + c * CH, CH)])
                pltpu.sync_copy(r2b[bb], r2_hbm.at[pl.ds(base + c * CH, CH)])

        c = N_CH - 1
        wait_g(0)
        pltpu.sync_copy(r1b[0], r1_hbm.at[pl.ds(base + c * CH, CH)])
        pltpu.sync_copy(r2b[0], r2_hbm.at[pl.ds(base + c * CH, CH)])

    return k(a, b, src, dst)


_RB = 2000
_EB = 2000


def _ln_relu(h, s, b):
    mu = jnp.mean(h, axis=-1, keepdims=True)
    d = h - mu
    var = jnp.mean(d * d, axis=-1, keepdims=True)
    return jnp.maximum(d * lax.rsqrt(var + 1e-5) * s + b, 0.0)


def _tc_encode(x, W_enc, b_enc, s0, b0):
    def body(x_ref, w_ref, be_ref, s_ref, b_ref, h_ref, t_ref):
        h = jnp.dot(x_ref[...], w_ref[...],
                    preferred_element_type=jnp.float32) + be_ref[...]
        h_ref[...] = h
        t_ref[...] = _ln_relu(h, s_ref[...], b_ref[...])

    full = pl.BlockSpec((H, H), lambda i: (0, 0))
    vec = pl.BlockSpec((1, H), lambda i: (0, 0))
    rows = pl.BlockSpec((_RB, H), lambda i: (i, 0))
    return pl.pallas_call(
        body,
        grid=(N // _RB,),
        in_specs=[rows, full, vec, vec, vec],
        out_specs=[rows, rows],
        out_shape=[jax.ShapeDtypeStruct((N, H), jnp.float32),
                   jax.ShapeDtypeStruct((N, H), jnp.float32)],
    )(x, W_enc, b_enc, s0, b0)


def _agg_from_parts(acc_ref, cnt_ref):
    a = acc_ref[0] + acc_ref[1]
    cnt = cnt_ref[0, :, :1] + cnt_ref[1, :, :1]
    inv = 1.0 / jnp.maximum(cnt, 1.0)
    eps = 1e-7 * (cnt > 0.0).astype(jnp.float32)
    return a * inv + eps


def _tc_layer(acc, cnt, h, Wl_i, bl_i, s_next, b_next):
    def body(acc_ref, cnt_ref, h_ref, w_ref, bv_ref, s_ref, b_ref,
             h_out, t_out):
        agg = _agg_from_parts(acc_ref, cnt_ref)
        hn = h_ref[...] + jnp.dot(agg, w_ref[...],
                                  preferred_element_type=jnp.float32) + bv_ref[...]
        h_out[...] = hn
        t_out[...] = _ln_relu(hn, s_ref[...], b_ref[...])

    rows = pl.BlockSpec((_RB, H), lambda i: (i, 0))
    acc_spec = pl.BlockSpec((NC, _RB, H), lambda i: (0, i, 0))
    cnt_spec = pl.BlockSpec((NC, _RB, H), lambda i: (0, i, 0))
    full = pl.BlockSpec((H, H), lambda i: (0, 0))
    vec = pl.BlockSpec((1, H), lambda i: (0, 0))
    return pl.pallas_call(
        body,
        grid=(N // _RB,),
        in_specs=[acc_spec, cnt_spec, rows, full, vec, vec, vec],
        out_specs=[rows, rows],
        out_shape=[jax.ShapeDtypeStruct((N, H), jnp.float32),
                   jax.ShapeDtypeStruct((N, H), jnp.float32)],
    )(acc, cnt, h, Wl_i, bl_i, s_next, b_next)


def _tc_final_proj(acc, cnt, h, Wl_i, bl_i, sf, bf, W1a, W1b, b1):
    def body(acc_ref, cnt_ref, h_ref, w_ref, bv_ref, s_ref, b_ref,
             w1a_ref, w1b_ref, b1_ref, a_out, b_out):
        agg = _agg_from_parts(acc_ref, cnt_ref)
        hn = h_ref[...] + jnp.dot(agg, w_ref[...],
                                  preferred_element_type=jnp.float32) + bv_ref[...]
        hf = _ln_relu(hn, s_ref[...], b_ref[...])
        a_out[...] = jnp.dot(hf, w1a_ref[...],
                             preferred_element_type=jnp.float32) + b1_ref[...]
        b_out[...] = jnp.dot(hf, w1b_ref[...],
                             preferred_element_type=jnp.float32)

    rows = pl.BlockSpec((_RB, H), lambda i: (i, 0))
    acc_spec = pl.BlockSpec((NC, _RB, H), lambda i: (0, i, 0))
    cnt_spec = pl.BlockSpec((NC, _RB, H), lambda i: (0, i, 0))
    full = pl.BlockSpec((H, H), lambda i: (0, 0))
    vec = pl.BlockSpec((1, H), lambda i: (0, 0))
    return pl.pallas_call(
        body,
        grid=(N // _RB,),
        in_specs=[acc_spec, cnt_spec, rows, full, vec, vec, vec,
                  full, full, vec],
        out_specs=[rows, rows],
        out_shape=[jax.ShapeDtypeStruct((N, H), jnp.float32),
                   jax.ShapeDtypeStruct((N, H), jnp.float32)],
    )(acc, cnt, h, Wl_i, bl_i, sf, bf, W1a, W1b, b1)


def _tc_mlp(r1, r2, W2, b2):
    def body(r1_ref, r2_ref, w_ref, bv_ref, o_ref):
        r = jnp.maximum(r1_ref[...] + r2_ref[...], 0.0)
        o_ref[...] = jnp.dot(r, w_ref[...],
                             preferred_element_type=jnp.float32) + bv_ref[...]

    rows = pl.BlockSpec((_EB, H), lambda i: (i, 0))
    full = pl.BlockSpec((H, H), lambda i: (0, 0))
    vec = pl.BlockSpec((1, H), lambda i: (0, 0))
    return pl.pallas_call(
        body,
        grid=(E // _EB,),
        in_specs=[rows, rows, full, vec],
        out_specs=rows,
        out_shape=jax.ShapeDtypeStruct((E, H), jnp.float32),
    )(r1, r2, W2, b2)


def kernel(x, edge_index, W_enc, b_enc, Wl, bl, ln_s, ln_b, lnf_s, lnf_b,
           W1, b1, W2, b2):
    L = Wl.shape[0]
    src = edge_index[0]
    dst = edge_index[1]
    zeros = jnp.zeros((N, H), jnp.float32)
    ones = jnp.ones((CH, H), jnp.float32)
    row = lambda v: v.reshape(1, -1)

    cnt = _sc_count(dst, ones, zeros).reshape(NC, N, H)
    h, t = _tc_encode(x, W_enc, row(b_enc), row(ln_s[0]), row(ln_b[0]))
    for i in range(L):
        acc = _sc_segsum(t, src, dst, zeros).reshape(NC, N, H)
        if i + 1 < L:
            h, t = _tc_layer(acc, cnt, h, Wl[i], row(bl[i]),
                             row(ln_s[i + 1]), row(ln_b[i + 1]))
        else:
            a_tab, b_tab = _tc_final_proj(acc, cnt, h, Wl[i], row(bl[i]),
                                          row(lnf_s), row(lnf_b),
                                          W1[:H], W1[H:], row(b1))
    r1, r2 = _sc_gather2(a_tab, b_tab, src, dst)
    return _tc_mlp(r1, r2, W2, row(b2))


# speedup vs baseline: 10.4577x; 1.0383x over previous
import functools

import jax
import jax.numpy as jnp
from jax import lax
from jax.experimental import pallas as pl
from jax.experimental.pallas import tpu as pltpu
from jax.experimental.pallas import tpu_sc as plsc

N = 10000
E = 320000
H = 128
NC = 2
NS = 16
NW = NC * NS
PER_W = E // NW
CH = 80
N_CH = PER_W // CH
ST = 624
REM = N - NS * ST

_sc_mesh = plsc.VectorSubcoreMesh(core_axis_name="c", subcore_axis_name="s",
                                  num_cores=NC, num_subcores=NS)


ACC_R = N + NS


def _acc_init(z_hbm, acc, sid):
    pltpu.sync_copy(z_hbm.at[pl.ds(sid * ST, ST)], acc.at[pl.ds(sid * ST, ST)])

    @pl.when(sid == NS - 1)
    def _():
        pltpu.sync_copy(z_hbm.at[pl.ds(NS * ST, REM)],
                        acc.at[pl.ds(NS * ST, REM)])


def _acc_dump(acc, out_hbm, cid, sid):
    pltpu.sync_copy(acc.at[pl.ds(sid * ST, ST)],
                    out_hbm.at[pl.ds(cid * N + sid * ST, ST)])

    @pl.when(sid == NS - 1)
    def _():
        pltpu.sync_copy(acc.at[pl.ds(NS * ST, REM)],
                        out_hbm.at[pl.ds(cid * N + NS * ST, REM)])


def _sc_segsum(t, src, dst, zeros):
    @functools.partial(
        pl.kernel,
        out_type=jax.ShapeDtypeStruct((NC * N, H), jnp.float32),
        mesh=_sc_mesh,
        scratch_types=(
            [pltpu.VMEM((CH,), jnp.int32)] * 4
            + [pltpu.VMEM((CH, H), jnp.float32)] * 2
            + [pltpu.VMEM_SHARED((ACC_R, H), jnp.float32)]
            + [pltpu.SemaphoreType.DMA] * 2
        ),
    )
    def k(t_hbm, src_hbm, dst_hbm, z_hbm, out_hbm,
          si0, si1, di0, di1, r0, r1, acc, sg0, sg1):
        sidx = (si0, si1)
        didx = (di0, di1)
        rows = (r0, r1)
        sg = (sg0, sg1)
        cid = lax.axis_index("c")
        sid = lax.axis_index("s")
        _acc_init(z_hbm, acc, sid)
        plsc.subcore_barrier()
        base = (cid * NS + sid) * PER_W

        def loadidx(c, b):
            pltpu.sync_copy(src_hbm.at[pl.ds(base + c * CH, CH)], sidx[b])
            pltpu.sync_copy(dst_hbm.at[pl.ds(base + c * CH, CH)], didx[b])

        def wait_g(b):
            pltpu.make_async_copy(t_hbm.at[sidx[0]], rows[b], sg[b]).wait()

        loadidx(0, 0)
        pltpu.async_copy(t_hbm.at[sidx[0]], rows[0], sg[0])

        @pl.loop(0, (N_CH - 1) // 2)
        def _(g):
            for b in range(2):
                c = 2 * g + b
                loadidx(c + 1, 1 - b)
                pltpu.async_copy(t_hbm.at[sidx[1 - b]], rows[1 - b],
                                 sg[1 - b])
                wait_g(b)
                pltpu.sync_copy(rows[b], acc.at[didx[b]], add=True)

        wait_g(0)
        pltpu.sync_copy(rows[0], acc.at[didx[0]], add=True)
        plsc.subcore_barrier()
        _acc_dump(acc, out_hbm, cid, sid)

    return k(t, src, dst, zeros)


def _sc_count(dst, ones, zeros):
    @functools.partial(
        pl.kernel,
        out_type=jax.ShapeDtypeStruct((NC * N, H), jnp.float32),
        mesh=_sc_mesh,
        scratch_types=[
            pltpu.VMEM((CH,), jnp.int32),
            pltpu.VMEM((CH, H), jnp.float32),
            pltpu.VMEM_SHARED((ACC_R, H), jnp.float32),
        ],
    )
    def k(dst_hbm, ones_hbm, z_hbm, out_hbm, didx, ones_v, acc):
        cid = lax.axis_index("c")
        sid = lax.axis_index("s")
        pltpu.sync_copy(ones_hbm, ones_v)
        _acc_init(z_hbm, acc, sid)
        plsc.subcore_barrier()
        base = (cid * NS + sid) * PER_W

        @pl.loop(0, N_CH)
        def _(c):
            pltpu.sync_copy(dst_hbm.at[pl.ds(base + c * CH, CH)], didx)
            pltpu.sync_copy(ones_v, acc.at[didx], add=True)

        plsc.subcore_barrier()
        _acc_dump(acc, out_hbm, cid, sid)

    return k(dst, ones, zeros)


def _sc_gather2(a, b, src, dst):
    @functools.partial(
        pl.kernel,
        out_type=[jax.ShapeDtypeStruct((E, H), jnp.float32),
                  jax.ShapeDtypeStruct((E, H), jnp.float32)],
        mesh=_sc_mesh,
        scratch_types=(
            [pltpu.VMEM((CH,), jnp.int32)] * 4
            + [pltpu.VMEM((CH, H), jnp.float32)] * 4
            + [pltpu.SemaphoreType.DMA] * 4
        ),
    )
    def k(a_hbm, b_hbm, src_hbm, dst_hbm, r1_hbm, r2_hbm,
          si0, si1, di0, di1, p0, p1, q0, q1, sg0, sg1, sw0, sw1):
        sidx = (si0, si1)
        didx = (di0, di1)
        r1b = (p0, p1)
        r2b = (q0, q1)
        sg = (sg0, sg1)
        sw = (sw0, sw1)
        cid = lax.axis_index("c")
        sid = lax.axis_index("s")
        base = (cid * NS + sid) * PER_W

        def loadidx(c, bb):
            pltpu.sync_copy(src_hbm.at[pl.ds(base + c * CH, CH)], sidx[bb])
            pltpu.sync_copy(dst_hbm.at[pl.ds(base + c * CH, CH)], didx[bb])

        def fire(bb):
            pltpu.async_copy(a_hbm.at[sidx[bb]], r1b[bb], sg[bb])
            pltpu.async_copy(b_hbm.at[didx[bb]], r2b[bb], sg[bb])

        def wait_g(bb):
            pltpu.make_async_copy(a_hbm.at[sidx[0]], r1b[bb], sg[bb]).wait()
            pltpu.make_async_copy(b_hbm.at[didx[0]], r2b[bb], sg[bb]).wait()

        def fire_w(c, bb):
            pltpu.async_copy(r1b[bb], r1_hbm.at[pl.ds(base + c * CH, CH)],
                             sw[bb])
            pltpu.async_copy(r2b[bb], r2_hbm.at[pl.ds(base + c * CH, CH)],
                             sw[bb])

        def wait_w(bb):
            pltpu.make_async_copy(r1b[bb], r1_hbm.at[pl.ds(base, CH)],
                                  sw[bb]).wait()
            pltpu.make_async_copy(r2b[bb], r2_hbm.at[pl.ds(base, CH)],
                                  sw[bb]).wait()

        loadidx(0, 0)
        fire(0)
        loadidx(1, 1)
        fire(1)
        wait_g(0)
        fire_w(0, 0)
        loadidx(2, 0)
        wait_w(0)
        fire(0)
        wait_g(1)
        fire_w(1, 1)

        @pl.loop(0, (N_CH - 3) // 2)
        def _(g):
            for bb in range(2):
                c = 2 * g + 2 + bb
                loadidx(c + 1, 1 - bb)
                wait_w(1 - bb)
                fire(1 - bb)
                wait_g(bb)
                fire_w(c, bb)

        c = N_CH - 1
        wait_g(0)
        fire_w(c, 0)
        wait_w(1)
        wait_w(0)

    return k(a, b, src, dst)


_RB = 2000
_EB = 2000


def _ln_relu(h, s, b):
    mu = jnp.mean(h, axis=-1, keepdims=True)
    d = h - mu
    var = jnp.mean(d * d, axis=-1, keepdims=True)
    return jnp.maximum(d * lax.rsqrt(var + 1e-5) * s + b, 0.0)


def _tc_encode(x, W_enc, b_enc, s0, b0):
    def body(x_ref, w_ref, be_ref, s_ref, b_ref, h_ref, t_ref):
        h = jnp.dot(x_ref[...], w_ref[...],
                    preferred_element_type=jnp.float32) + be_ref[...]
        h_ref[...] = h
        t_ref[...] = _ln_relu(h, s_ref[...], b_ref[...])

    full = pl.BlockSpec((H, H), lambda i: (0, 0))
    vec = pl.BlockSpec((1, H), lambda i: (0, 0))
    rows = pl.BlockSpec((_RB, H), lambda i: (i, 0))
    return pl.pallas_call(
        body,
        grid=(N // _RB,),
        in_specs=[rows, full, vec, vec, vec],
        out_specs=[rows, rows],
        out_shape=[jax.ShapeDtypeStruct((N, H), jnp.float32),
                   jax.ShapeDtypeStruct((N, H), jnp.float32)],
    )(x, W_enc, b_enc, s0, b0)


def _agg_from_parts(acc_ref, cnt_ref):
    a = acc_ref[0] + acc_ref[1]
    cnt = cnt_ref[0, :, :1] + cnt_ref[1, :, :1]
    inv = 1.0 / jnp.maximum(cnt, 1.0)
    eps = 1e-7 * (cnt > 0.0).astype(jnp.float32)
    return a * inv + eps


def _tc_layer(acc, cnt, h, Wl_i, bl_i, s_next, b_next):
    def body(acc_ref, cnt_ref, h_ref, w_ref, bv_ref, s_ref, b_ref,
             h_out, t_out):
        agg = _agg_from_parts(acc_ref, cnt_ref)
        hn = h_ref[...] + jnp.dot(agg, w_ref[...],
                                  preferred_element_type=jnp.float32) + bv_ref[...]
        h_out[...] = hn
        t_out[...] = _ln_relu(hn, s_ref[...], b_ref[...])

    rows = pl.BlockSpec((_RB, H), lambda i: (i, 0))
    acc_spec = pl.BlockSpec((NC, _RB, H), lambda i: (0, i, 0))
    cnt_spec = pl.BlockSpec((NC, _RB, H), lambda i: (0, i, 0))
    full = pl.BlockSpec((H, H), lambda i: (0, 0))
    vec = pl.BlockSpec((1, H), lambda i: (0, 0))
    return pl.pallas_call(
        body,
        grid=(N // _RB,),
        in_specs=[acc_spec, cnt_spec, rows, full, vec, vec, vec],
        out_specs=[rows, rows],
        out_shape=[jax.ShapeDtypeStruct((N, H), jnp.float32),
                   jax.ShapeDtypeStruct((N, H), jnp.float32)],
    )(acc, cnt, h, Wl_i, bl_i, s_next, b_next)


def _tc_final_proj(acc, cnt, h, Wl_i, bl_i, sf, bf, W1a, W1b, b1):
    def body(acc_ref, cnt_ref, h_ref, w_ref, bv_ref, s_ref, b_ref,
             w1a_ref, w1b_ref, b1_ref, a_out, b_out):
        agg = _agg_from_parts(acc_ref, cnt_ref)
        hn = h_ref[...] + jnp.dot(agg, w_ref[...],
                                  preferred_element_type=jnp.float32) + bv_ref[...]
        hf = _ln_relu(hn, s_ref[...], b_ref[...])
        a_out[...] = jnp.dot(hf, w1a_ref[...],
                             preferred_element_type=jnp.float32) + b1_ref[...]
        b_out[...] = jnp.dot(hf, w1b_ref[...],
                             preferred_element_type=jnp.float32)

    rows = pl.BlockSpec((_RB, H), lambda i: (i, 0))
    acc_spec = pl.BlockSpec((NC, _RB, H), lambda i: (0, i, 0))
    cnt_spec = pl.BlockSpec((NC, _RB, H), lambda i: (0, i, 0))
    full = pl.BlockSpec((H, H), lambda i: (0, 0))
    vec = pl.BlockSpec((1, H), lambda i: (0, 0))
    return pl.pallas_call(
        body,
        grid=(N // _RB,),
        in_specs=[acc_spec, cnt_spec, rows, full, vec, vec, vec,
                  full, full, vec],
        out_specs=[rows, rows],
        out_shape=[jax.ShapeDtypeStruct((N, H), jnp.float32),
                   jax.ShapeDtypeStruct((N, H), jnp.float32)],
    )(acc, cnt, h, Wl_i, bl_i, sf, bf, W1a, W1b, b1)


def _tc_mlp(r1, r2, W2, b2):
    def body(r1_ref, r2_ref, w_ref, bv_ref, o_ref):
        r = jnp.maximum(r1_ref[...] + r2_ref[...], 0.0)
        o_ref[...] = jnp.dot(r, w_ref[...],
                             preferred_element_type=jnp.float32) + bv_ref[...]

    rows = pl.BlockSpec((_EB, H), lambda i: (i, 0))
    full = pl.BlockSpec((H, H), lambda i: (0, 0))
    vec = pl.BlockSpec((1, H), lambda i: (0, 0))
    return pl.pallas_call(
        body,
        grid=(E // _EB,),
        in_specs=[rows, rows, full, vec],
        out_specs=rows,
        out_shape=jax.ShapeDtypeStruct((E, H), jnp.float32),
    )(r1, r2, W2, b2)


def kernel(x, edge_index, W_enc, b_enc, Wl, bl, ln_s, ln_b, lnf_s, lnf_b,
           W1, b1, W2, b2):
    L = Wl.shape[0]
    src = edge_index[0]
    dst = edge_index[1]
    zeros = jnp.zeros((N, H), jnp.float32)
    ones = jnp.ones((CH, H), jnp.float32)
    row = lambda v: v.reshape(1, -1)

    cnt = _sc_count(dst, ones, zeros).reshape(NC, N, H)
    h, t = _tc_encode(x, W_enc, row(b_enc), row(ln_s[0]), row(ln_b[0]))
    for i in range(L):
        acc = _sc_segsum(t, src, dst, zeros).reshape(NC, N, H)
        if i + 1 < L:
            h, t = _tc_layer(acc, cnt, h, Wl[i], row(bl[i]),
                             row(ln_s[i + 1]), row(ln_b[i + 1]))
        else:
            a_tab, b_tab = _tc_final_proj(acc, cnt, h, Wl[i], row(bl[i]),
                                          row(lnf_s), row(lnf_b),
                                          W1[:H], W1[H:], row(b1))
    r1, r2 = _sc_gather2(a_tab, b_tab, src, dst)
    return _tc_mlp(r1, r2, W2, row(b2))


# speedup vs baseline: 10.7846x; 1.0313x over previous
import functools

import jax
import jax.numpy as jnp
from jax import lax
from jax.experimental import pallas as pl
from jax.experimental.pallas import tpu as pltpu
from jax.experimental.pallas import tpu_sc as plsc

N = 10000
E = 320000
H = 128
NC = 2
NS = 16
NW = NC * NS
PER_W = E // NW
CH = 80
N_CH = PER_W // CH
ST = 624
REM = N - NS * ST

_sc_mesh = plsc.VectorSubcoreMesh(core_axis_name="c", subcore_axis_name="s",
                                  num_cores=NC, num_subcores=NS)


ACC_R = N + NS


def _acc_init(z_hbm, acc, sid):
    pltpu.sync_copy(z_hbm.at[pl.ds(sid * ST, ST)], acc.at[pl.ds(sid * ST, ST)])

    @pl.when(sid == NS - 1)
    def _():
        pltpu.sync_copy(z_hbm.at[pl.ds(NS * ST, REM)],
                        acc.at[pl.ds(NS * ST, REM)])


def _acc_dump(acc, out_hbm, cid, sid):
    pltpu.sync_copy(acc.at[pl.ds(sid * ST, ST)],
                    out_hbm.at[pl.ds(cid * N + sid * ST, ST)])

    @pl.when(sid == NS - 1)
    def _():
        pltpu.sync_copy(acc.at[pl.ds(NS * ST, REM)],
                        out_hbm.at[pl.ds(cid * N + NS * ST, REM)])


def _sc_segsum(t, src, dst, zeros):
    @functools.partial(
        pl.kernel,
        out_type=jax.ShapeDtypeStruct((NC * N, H), jnp.float32),
        mesh=_sc_mesh,
        scratch_types=(
            [pltpu.VMEM((CH,), jnp.int32)] * 4
            + [pltpu.VMEM((CH, H), jnp.float32)] * 2
            + [pltpu.VMEM_SHARED((ACC_R, H), jnp.float32)]
            + [pltpu.SemaphoreType.DMA] * 2
        ),
    )
    def k(t_hbm, src_hbm, dst_hbm, z_hbm, out_hbm,
          si0, si1, di0, di1, r0, r1, acc, sg0, sg1):
        sidx = (si0, si1)
        didx = (di0, di1)
        rows = (r0, r1)
        sg = (sg0, sg1)
        cid = lax.axis_index("c")
        sid = lax.axis_index("s")
        _acc_init(z_hbm, acc, sid)
        plsc.subcore_barrier()
        base = (cid * NS + sid) * PER_W

        def loadidx(c, b):
            pltpu.sync_copy(src_hbm.at[pl.ds(base + c * CH, CH)], sidx[b])
            pltpu.sync_copy(dst_hbm.at[pl.ds(base + c * CH, CH)], didx[b])

        def wait_g(b):
            pltpu.make_async_copy(t_hbm.at[sidx[0]], rows[b], sg[b]).wait()

        loadidx(0, 0)
        pltpu.async_copy(t_hbm.at[sidx[0]], rows[0], sg[0])

        @pl.loop(0, (N_CH - 1) // 2)
        def _(g):
            for b in range(2):
                c = 2 * g + b
                loadidx(c + 1, 1 - b)
                pltpu.async_copy(t_hbm.at[sidx[1 - b]], rows[1 - b],
                                 sg[1 - b])
                wait_g(b)
                pltpu.sync_copy(rows[b], acc.at[didx[b]], add=True)

        wait_g(0)
        pltpu.sync_copy(rows[0], acc.at[didx[0]], add=True)
        plsc.subcore_barrier()
        _acc_dump(acc, out_hbm, cid, sid)

    return k(t, src, dst, zeros)


def _sc_count(dst, ones, zeros):
    @functools.partial(
        pl.kernel,
        out_type=jax.ShapeDtypeStruct((NC * N, H), jnp.float32),
        mesh=_sc_mesh,
        scratch_types=[
            pltpu.VMEM((CH,), jnp.int32),
            pltpu.VMEM((CH, H), jnp.float32),
            pltpu.VMEM_SHARED((ACC_R, H), jnp.float32),
        ],
    )
    def k(dst_hbm, ones_hbm, z_hbm, out_hbm, didx, ones_v, acc):
        cid = lax.axis_index("c")
        sid = lax.axis_index("s")
        pltpu.sync_copy(ones_hbm, ones_v)
        _acc_init(z_hbm, acc, sid)
        plsc.subcore_barrier()
        base = (cid * NS + sid) * PER_W

        @pl.loop(0, N_CH)
        def _(c):
            pltpu.sync_copy(dst_hbm.at[pl.ds(base + c * CH, CH)], didx)
            pltpu.sync_copy(ones_v, acc.at[didx], add=True)

        plsc.subcore_barrier()
        _acc_dump(acc, out_hbm, cid, sid)

    return k(dst, ones, zeros)


def _sc_gather2(a, b, src, dst):
    @functools.partial(
        pl.kernel,
        out_type=[jax.ShapeDtypeStruct((E, H), jnp.float32),
                  jax.ShapeDtypeStruct((E, H), jnp.float32)],
        mesh=_sc_mesh,
        scratch_types=(
            [pltpu.VMEM((CH,), jnp.int32)] * 4
            + [pltpu.VMEM((CH, H), jnp.float32)] * 4
            + [pltpu.SemaphoreType.DMA] * 4
        ),
    )
    def k(a_hbm, b_hbm, src_hbm, dst_hbm, r1_hbm, r2_hbm,
          si0, si1, di0, di1, p0, p1, q0, q1, sg0, sg1, sw0, sw1):
        sidx = (si0, si1)
        didx = (di0, di1)
        r1b = (p0, p1)
        r2b = (q0, q1)
        sg = (sg0, sg1)
        sw = (sw0, sw1)
        cid = lax.axis_index("c")
        sid = lax.axis_index("s")
        base = (cid * NS + sid) * PER_W

        def loadidx(c, bb):
            pltpu.sync_copy(src_hbm.at[pl.ds(base + c * CH, CH)], sidx[bb])
            pltpu.sync_copy(dst_hbm.at[pl.ds(base + c * CH, CH)], didx[bb])

        def fire(bb):
            pltpu.async_copy(a_hbm.at[sidx[bb]], r1b[bb], sg[bb])
            pltpu.async_copy(b_hbm.at[didx[bb]], r2b[bb], sg[bb])

        def wait_g(bb):
            pltpu.make_async_copy(a_hbm.at[sidx[0]], r1b[bb], sg[bb]).wait()
            pltpu.make_async_copy(b_hbm.at[didx[0]], r2b[bb], sg[bb]).wait()

        def fire_w(c, bb):
            pltpu.async_copy(r1b[bb], r1_hbm.at[pl.ds(base + c * CH, CH)],
                             sw[bb])
            pltpu.async_copy(r2b[bb], r2_hbm.at[pl.ds(base + c * CH, CH)],
                             sw[bb])

        def wait_w(bb):
            pltpu.make_async_copy(r1b[bb], r1_hbm.at[pl.ds(base, CH)],
                                  sw[bb]).wait()
            pltpu.make_async_copy(r2b[bb], r2_hbm.at[pl.ds(base, CH)],
                                  sw[bb]).wait()

        loadidx(0, 0)
        fire(0)
        loadidx(1, 1)
        fire(1)
        wait_g(0)
        fire_w(0, 0)
        loadidx(2, 0)
        wait_w(0)
        fire(0)
        wait_g(1)
        fire_w(1, 1)

        @pl.loop(0, (N_CH - 3) // 2)
        def _(g):
            for bb in range(2):
                c = 2 * g + 2 + bb
                loadidx(c + 1, 1 - bb)
                wait_w(1 - bb)
                fire(1 - bb)
                wait_g(bb)
                fire_w(c, bb)

        c = N_CH - 1
        wait_g(0)
        fire_w(c, 0)
        wait_w(1)
        wait_w(0)

    return k(a, b, src, dst)


_RB = 2000
_EB = 4000


def _ln_relu(h, s, b):
    mu = jnp.mean(h, axis=-1, keepdims=True)
    d = h - mu
    var = jnp.mean(d * d, axis=-1, keepdims=True)
    return jnp.maximum(d * lax.rsqrt(var + 1e-5) * s + b, 0.0)


def _tc_encode(x, W_enc, b_enc, s0, b0):
    def body(x_ref, w_ref, be_ref, s_ref, b_ref, h_ref, t_ref):
        h = jnp.dot(x_ref[...], w_ref[...],
                    preferred_element_type=jnp.float32) + be_ref[...]
        h_ref[...] = h
        t_ref[...] = _ln_relu(h, s_ref[...], b_ref[...])

    full = pl.BlockSpec((H, H), lambda i: (0, 0))
    vec = pl.BlockSpec((1, H), lambda i: (0, 0))
    rows = pl.BlockSpec((_RB, H), lambda i: (i, 0))
    return pl.pallas_call(
        body,
        grid=(N // _RB,),
        in_specs=[rows, full, vec, vec, vec],
        out_specs=[rows, rows],
        out_shape=[jax.ShapeDtypeStruct((N, H), jnp.float32),
                   jax.ShapeDtypeStruct((N, H), jnp.float32)],
    )(x, W_enc, b_enc, s0, b0)


def _agg_from_parts(acc_ref, cnt_ref):
    a = acc_ref[0] + acc_ref[1]
    cnt = cnt_ref[0, :, :1] + cnt_ref[1, :, :1]
    inv = 1.0 / jnp.maximum(cnt, 1.0)
    eps = 1e-7 * (cnt > 0.0).astype(jnp.float32)
    return a * inv + eps


def _tc_layer(acc, cnt, h, Wl_i, bl_i, s_next, b_next):
    def body(acc_ref, cnt_ref, h_ref, w_ref, bv_ref, s_ref, b_ref,
             h_out, t_out):
        agg = _agg_from_parts(acc_ref, cnt_ref)
        hn = h_ref[...] + jnp.dot(agg, w_ref[...],
                                  preferred_element_type=jnp.float32) + bv_ref[...]
        h_out[...] = hn
        t_out[...] = _ln_relu(hn, s_ref[...], b_ref[...])

    rows = pl.BlockSpec((_RB, H), lambda i: (i, 0))
    acc_spec = pl.BlockSpec((NC, _RB, H), lambda i: (0, i, 0))
    cnt_spec = pl.BlockSpec((NC, _RB, H), lambda i: (0, i, 0))
    full = pl.BlockSpec((H, H), lambda i: (0, 0))
    vec = pl.BlockSpec((1, H), lambda i: (0, 0))
    return pl.pallas_call(
        body,
        grid=(N // _RB,),
        in_specs=[acc_spec, cnt_spec, rows, full, vec, vec, vec],
        out_specs=[rows, rows],
        out_shape=[jax.ShapeDtypeStruct((N, H), jnp.float32),
                   jax.ShapeDtypeStruct((N, H), jnp.float32)],
    )(acc, cnt, h, Wl_i, bl_i, s_next, b_next)


def _tc_final_proj(acc, cnt, h, Wl_i, bl_i, sf, bf, W1a, W1b, b1):
    def body(acc_ref, cnt_ref, h_ref, w_ref, bv_ref, s_ref, b_ref,
             w1a_ref, w1b_ref, b1_ref, a_out, b_out):
        agg = _agg_from_parts(acc_ref, cnt_ref)
        hn = h_ref[...] + jnp.dot(agg, w_ref[...],
                                  preferred_element_type=jnp.float32) + bv_ref[...]
        hf = _ln_relu(hn, s_ref[...], b_ref[...])
        a_out[...] = jnp.dot(hf, w1a_ref[...],
                             preferred_element_type=jnp.float32) + b1_ref[...]
        b_out[...] = jnp.dot(hf, w1b_ref[...],
                             preferred_element_type=jnp.float32)

    rows = pl.BlockSpec((_RB, H), lambda i: (i, 0))
    acc_spec = pl.BlockSpec((NC, _RB, H), lambda i: (0, i, 0))
    cnt_spec = pl.BlockSpec((NC, _RB, H), lambda i: (0, i, 0))
    full = pl.BlockSpec((H, H), lambda i: (0, 0))
    vec = pl.BlockSpec((1, H), lambda i: (0, 0))
    return pl.pallas_call(
        body,
        grid=(N // _RB,),
        in_specs=[acc_spec, cnt_spec, rows, full, vec, vec, vec,
                  full, full, vec],
        out_specs=[rows, rows],
        out_shape=[jax.ShapeDtypeStruct((N, H), jnp.float32),
                   jax.ShapeDtypeStruct((N, H), jnp.float32)],
    )(acc, cnt, h, Wl_i, bl_i, sf, bf, W1a, W1b, b1)


def _tc_mlp(r1, r2, W2, b2):
    def body(r1_ref, r2_ref, w_ref, bv_ref, o_ref):
        r = jnp.maximum(r1_ref[...] + r2_ref[...], 0.0)
        o_ref[...] = jnp.dot(r, w_ref[...],
                             preferred_element_type=jnp.float32) + bv_ref[...]

    rows = pl.BlockSpec((_EB, H), lambda i: (i, 0))
    full = pl.BlockSpec((H, H), lambda i: (0, 0))
    vec = pl.BlockSpec((1, H), lambda i: (0, 0))
    return pl.pallas_call(
        body,
        grid=(E // _EB,),
        in_specs=[rows, rows, full, vec],
        out_specs=rows,
        out_shape=jax.ShapeDtypeStruct((E, H), jnp.float32),
    )(r1, r2, W2, b2)


def kernel(x, edge_index, W_enc, b_enc, Wl, bl, ln_s, ln_b, lnf_s, lnf_b,
           W1, b1, W2, b2):
    L = Wl.shape[0]
    src = edge_index[0]
    dst = edge_index[1]
    zeros = jnp.zeros((N, H), jnp.float32)
    ones = jnp.ones((CH, H), jnp.float32)
    row = lambda v: v.reshape(1, -1)

    cnt = _sc_count(dst, ones, zeros).reshape(NC, N, H)
    h, t = _tc_encode(x, W_enc, row(b_enc), row(ln_s[0]), row(ln_b[0]))
    for i in range(L):
        acc = _sc_segsum(t, src, dst, zeros).reshape(NC, N, H)
        if i + 1 < L:
            h, t = _tc_layer(acc, cnt, h, Wl[i], row(bl[i]),
                             row(ln_s[i + 1]), row(ln_b[i + 1]))
        else:
            a_tab, b_tab = _tc_final_proj(acc, cnt, h, Wl[i], row(bl[i]),
                                          row(lnf_s), row(lnf_b),
                                          W1[:H], W1[H:], row(b1))
    r1, r2 = _sc_gather2(a_tab, b_tab, src, dst)
    return _tc_mlp(r1, r2, W2, row(b2))


# speedup vs baseline: 10.8873x; 1.0095x over previous
import functools

import jax
import jax.numpy as jnp
from jax import lax
from jax.experimental import pallas as pl
from jax.experimental.pallas import tpu as pltpu
from jax.experimental.pallas import tpu_sc as plsc

N = 10000
E = 320000
H = 128
NC = 2
NS = 16
NW = NC * NS
PER_W = E // NW
CH = 80
N_CH = PER_W // CH
ST = 624
REM = N - NS * ST

_sc_mesh = plsc.VectorSubcoreMesh(core_axis_name="c", subcore_axis_name="s",
                                  num_cores=NC, num_subcores=NS)


ACC_R = N + NS


def _acc_init(z_hbm, acc, sid):
    pltpu.sync_copy(z_hbm.at[pl.ds(sid * ST, ST)], acc.at[pl.ds(sid * ST, ST)])

    @pl.when(sid == NS - 1)
    def _():
        pltpu.sync_copy(z_hbm.at[pl.ds(NS * ST, REM)],
                        acc.at[pl.ds(NS * ST, REM)])


def _acc_dump(acc, out_hbm, cid, sid):
    pltpu.sync_copy(acc.at[pl.ds(sid * ST, ST)],
                    out_hbm.at[pl.ds(cid * N + sid * ST, ST)])

    @pl.when(sid == NS - 1)
    def _():
        pltpu.sync_copy(acc.at[pl.ds(NS * ST, REM)],
                        out_hbm.at[pl.ds(cid * N + NS * ST, REM)])


def _sc_segsum(t, src, dst, zeros):
    @functools.partial(
        pl.kernel,
        out_type=jax.ShapeDtypeStruct((NC * N, H), jnp.float32),
        mesh=_sc_mesh,
        scratch_types=(
            [pltpu.VMEM((CH,), jnp.int32)] * 4
            + [pltpu.VMEM((CH, H), jnp.float32)] * 2
            + [pltpu.VMEM_SHARED((ACC_R, H), jnp.float32)]
            + [pltpu.SemaphoreType.DMA] * 2
        ),
    )
    def k(t_hbm, src_hbm, dst_hbm, z_hbm, out_hbm,
          si0, si1, di0, di1, r0, r1, acc, sg0, sg1):
        sidx = (si0, si1)
        didx = (di0, di1)
        rows = (r0, r1)
        sg = (sg0, sg1)
        cid = lax.axis_index("c")
        sid = lax.axis_index("s")
        _acc_init(z_hbm, acc, sid)
        plsc.subcore_barrier()
        base = (cid * NS + sid) * PER_W

        def loadidx(c, b):
            pltpu.sync_copy(src_hbm.at[pl.ds(base + c * CH, CH)], sidx[b])
            pltpu.sync_copy(dst_hbm.at[pl.ds(base + c * CH, CH)], didx[b])

        def wait_g(b):
            pltpu.make_async_copy(t_hbm.at[sidx[0]], rows[b], sg[b]).wait()

        loadidx(0, 0)
        pltpu.async_copy(t_hbm.at[sidx[0]], rows[0], sg[0])

        @pl.loop(0, (N_CH - 1) // 2)
        def _(g):
            for b in range(2):
                c = 2 * g + b
                loadidx(c + 1, 1 - b)
                pltpu.async_copy(t_hbm.at[sidx[1 - b]], rows[1 - b],
                                 sg[1 - b])
                wait_g(b)
                pltpu.sync_copy(rows[b], acc.at[didx[b]], add=True)

        wait_g(0)
        pltpu.sync_copy(rows[0], acc.at[didx[0]], add=True)
        plsc.subcore_barrier()
        _acc_dump(acc, out_hbm, cid, sid)

    return k(t, src, dst, zeros)


def _sc_count(dst, ones, zeros):
    @functools.partial(
        pl.kernel,
        out_type=jax.ShapeDtypeStruct((NC * N, H), jnp.float32),
        mesh=_sc_mesh,
        scratch_types=[
            pltpu.VMEM((CH,), jnp.int32),
            pltpu.VMEM((CH, H), jnp.float32),
            pltpu.VMEM_SHARED((ACC_R, H), jnp.float32),
        ],
    )
    def k(dst_hbm, ones_hbm, z_hbm, out_hbm, didx, ones_v, acc):
        cid = lax.axis_index("c")
        sid = lax.axis_index("s")
        pltpu.sync_copy(ones_hbm, ones_v)
        _acc_init(z_hbm, acc, sid)
        plsc.subcore_barrier()
        base = (cid * NS + sid) * PER_W

        @pl.loop(0, N_CH)
        def _(c):
            pltpu.sync_copy(dst_hbm.at[pl.ds(base + c * CH, CH)], didx)
            pltpu.sync_copy(ones_v, acc.at[didx], add=True)

        plsc.subcore_barrier()
        _acc_dump(acc, out_hbm, cid, sid)

    return k(dst, ones, zeros)


def _sc_gather2(a, b, src, dst):
    @functools.partial(
        pl.kernel,
        out_type=[jax.ShapeDtypeStruct((E, H), jnp.float32),
                  jax.ShapeDtypeStruct((E, H), jnp.float32)],
        mesh=_sc_mesh,
        scratch_types=(
            [pltpu.VMEM((CH,), jnp.int32)] * 4
            + [pltpu.VMEM((CH, H), jnp.float32)] * 4
            + [pltpu.SemaphoreType.DMA] * 4
        ),
    )
    def k(a_hbm, b_hbm, src_hbm, dst_hbm, r1_hbm, r2_hbm,
          si0, si1, di0, di1, p0, p1, q0, q1, sg0, sg1, sw0, sw1):
        sidx = (si0, si1)
        didx = (di0, di1)
        r1b = (p0, p1)
        r2b = (q0, q1)
        sg = (sg0, sg1)
        sw = (sw0, sw1)
        cid = lax.axis_index("c")
        sid = lax.axis_index("s")
        base = (cid * NS + sid) * PER_W

        def loadidx(c, bb):
            pltpu.sync_copy(src_hbm.at[pl.ds(base + c * CH, CH)], sidx[bb])
            pltpu.sync_copy(dst_hbm.at[pl.ds(base + c * CH, CH)], didx[bb])

        def fire(bb):
            pltpu.async_copy(a_hbm.at[sidx[bb]], r1b[bb], sg[bb])
            pltpu.async_copy(b_hbm.at[didx[bb]], r2b[bb], sg[bb])

        def wait_g(bb):
            pltpu.make_async_copy(a_hbm.at[sidx[0]], r1b[bb], sg[bb]).wait()
            pltpu.make_async_copy(b_hbm.at[didx[0]], r2b[bb], sg[bb]).wait()

        def fire_w(c, bb):
            pltpu.async_copy(r1b[bb], r1_hbm.at[pl.ds(base + c * CH, CH)],
                             sw[bb])
            pltpu.async_copy(r2b[bb], r2_hbm.at[pl.ds(base + c * CH, CH)],
                             sw[bb])

        def wait_w(bb):
            pltpu.make_async_copy(r1b[bb], r1_hbm.at[pl.ds(base, CH)],
                                  sw[bb]).wait()
            pltpu.make_async_copy(r2b[bb], r2_hbm.at[pl.ds(base, CH)],
                                  sw[bb]).wait()

        loadidx(0, 0)
        fire(0)
        loadidx(1, 1)
        fire(1)
        wait_g(0)
        fire_w(0, 0)
        loadidx(2, 0)
        wait_w(0)
        fire(0)
        wait_g(1)
        fire_w(1, 1)

        @pl.loop(0, (N_CH - 3) // 2)
        def _(g):
            for bb in range(2):
                c = 2 * g + 2 + bb
                loadidx(c + 1, 1 - bb)
                wait_w(1 - bb)
                fire(1 - bb)
                wait_g(bb)
                fire_w(c, bb)

        c = N_CH - 1
        wait_g(0)
        fire_w(c, 0)
        wait_w(1)
        wait_w(0)

    return k(a, b, src, dst)


_RB = 5000
_EB = 8000


def _ln_relu(h, s, b):
    mu = jnp.mean(h, axis=-1, keepdims=True)
    d = h - mu
    var = jnp.mean(d * d, axis=-1, keepdims=True)
    return jnp.maximum(d * lax.rsqrt(var + 1e-5) * s + b, 0.0)


def _tc_encode(x, W_enc, b_enc, s0, b0):
    def body(x_ref, w_ref, be_ref, s_ref, b_ref, h_ref, t_ref):
        h = jnp.dot(x_ref[...], w_ref[...],
                    preferred_element_type=jnp.float32) + be_ref[...]
        h_ref[...] = h
        t_ref[...] = _ln_relu(h, s_ref[...], b_ref[...])

    full = pl.BlockSpec((H, H), lambda i: (0, 0))
    vec = pl.BlockSpec((1, H), lambda i: (0, 0))
    rows = pl.BlockSpec((_RB, H), lambda i: (i, 0))
    return pl.pallas_call(
        body,
        grid=(N // _RB,),
        in_specs=[rows, full, vec, vec, vec],
        out_specs=[rows, rows],
        out_shape=[jax.ShapeDtypeStruct((N, H), jnp.float32),
                   jax.ShapeDtypeStruct((N, H), jnp.float32)],
    )(x, W_enc, b_enc, s0, b0)


def _agg_from_parts(acc_ref, cnt_ref):
    a = acc_ref[0] + acc_ref[1]
    cnt = cnt_ref[0, :, :1] + cnt_ref[1, :, :1]
    inv = 1.0 / jnp.maximum(cnt, 1.0)
    eps = 1e-7 * (cnt > 0.0).astype(jnp.float32)
    return a * inv + eps


def _tc_layer(acc, cnt, h, Wl_i, bl_i, s_next, b_next):
    def body(acc_ref, cnt_ref, h_ref, w_ref, bv_ref, s_ref, b_ref,
             h_out, t_out):
        agg = _agg_from_parts(acc_ref, cnt_ref)
        hn = h_ref[...] + jnp.dot(agg, w_ref[...],
                                  preferred_element_type=jnp.float32) + bv_ref[...]
        h_out[...] = hn
        t_out[...] = _ln_relu(hn, s_ref[...], b_ref[...])

    rows = pl.BlockSpec((_RB, H), lambda i: (i, 0))
    acc_spec = pl.BlockSpec((NC, _RB, H), lambda i: (0, i, 0))
    cnt_spec = pl.BlockSpec((NC, _RB, H), lambda i: (0, i, 0))
    full = pl.BlockSpec((H, H), lambda i: (0, 0))
    vec = pl.BlockSpec((1, H), lambda i: (0, 0))
    return pl.pallas_call(
        body,
        grid=(N // _RB,),
        in_specs=[acc_spec, cnt_spec, rows, full, vec, vec, vec],
        out_specs=[rows, rows],
        out_shape=[jax.ShapeDtypeStruct((N, H), jnp.float32),
                   jax.ShapeDtypeStruct((N, H), jnp.float32)],
    )(acc, cnt, h, Wl_i, bl_i, s_next, b_next)


def _tc_final_proj(acc, cnt, h, Wl_i, bl_i, sf, bf, W1a, W1b, b1):
    def body(acc_ref, cnt_ref, h_ref, w_ref, bv_ref, s_ref, b_ref,
             w1a_ref, w1b_ref, b1_ref, a_out, b_out):
        agg = _agg_from_parts(acc_ref, cnt_ref)
        hn = h_ref[...] + jnp.dot(agg, w_ref[...],
                                  preferred_element_type=jnp.float32) + bv_ref[...]
        hf = _ln_relu(hn, s_ref[...], b_ref[...])
        a_out[...] = jnp.dot(hf, w1a_ref[...],
                             preferred_element_type=jnp.float32) + b1_ref[...]
        b_out[...] = jnp.dot(hf, w1b_ref[...],
                             preferred_element_type=jnp.float32)

    rows = pl.BlockSpec((_RB, H), lambda i: (i, 0))
    acc_spec = pl.BlockSpec((NC, _RB, H), lambda i: (0, i, 0))
    cnt_spec = pl.BlockSpec((NC, _RB, H), lambda i: (0, i, 0))
    full = pl.BlockSpec((H, H), lambda i: (0, 0))
    vec = pl.BlockSpec((1, H), lambda i: (0, 0))
    return pl.pallas_call(
        body,
        grid=(N // _RB,),
        in_specs=[acc_spec, cnt_spec, rows, full, vec, vec, vec,
                  full, full, vec],
        out_specs=[rows, rows],
        out_shape=[jax.ShapeDtypeStruct((N, H), jnp.float32),
                   jax.ShapeDtypeStruct((N, H), jnp.float32)],
    )(acc, cnt, h, Wl_i, bl_i, sf, bf, W1a, W1b, b1)


def _tc_mlp(r1, r2, W2, b2):
    def body(r1_ref, r2_ref, w_ref, bv_ref, o_ref):
        r = jnp.maximum(r1_ref[...] + r2_ref[...], 0.0)
        o_ref[...] = jnp.dot(r, w_ref[...],
                             preferred_element_type=jnp.float32) + bv_ref[...]

    rows = pl.BlockSpec((_EB, H), lambda i: (i, 0))
    full = pl.BlockSpec((H, H), lambda i: (0, 0))
    vec = pl.BlockSpec((1, H), lambda i: (0, 0))
    return pl.pallas_call(
        body,
        grid=(E // _EB,),
        in_specs=[rows, rows, full, vec],
        out_specs=rows,
        out_shape=jax.ShapeDtypeStruct((E, H), jnp.float32),
    )(r1, r2, W2, b2)


def kernel(x, edge_index, W_enc, b_enc, Wl, bl, ln_s, ln_b, lnf_s, lnf_b,
           W1, b1, W2, b2):
    L = Wl.shape[0]
    src = edge_index[0]
    dst = edge_index[1]
    zeros = jnp.zeros((N, H), jnp.float32)
    ones = jnp.ones((CH, H), jnp.float32)
    row = lambda v: v.reshape(1, -1)

    cnt = _sc_count(dst, ones, zeros).reshape(NC, N, H)
    h, t = _tc_encode(x, W_enc, row(b_enc), row(ln_s[0]), row(ln_b[0]))
    for i in range(L):
        acc = _sc_segsum(t, src, dst, zeros).reshape(NC, N, H)
        if i + 1 < L:
            h, t = _tc_layer(acc, cnt, h, Wl[i], row(bl[i]),
                             row(ln_s[i + 1]), row(ln_b[i + 1]))
        else:
            a_tab, b_tab = _tc_final_proj(acc, cnt, h, Wl[i], row(bl[i]),
                                          row(lnf_s), row(lnf_b),
                                          W1[:H], W1[H:], row(b1))
    r1, r2 = _sc_gather2(a_tab, b_tab, src, dst)
    return _tc_mlp(r1, r2, W2, row(b2))
